# baseline (speedup 1.0000x reference)
import dataclasses
import functools

import jax
import jax.numpy as jnp
from jax import lax
from jax.experimental import pallas as pl
from jax.experimental.pallas import tpu as pltpu
from jax.experimental.pallas import tpu_sc as plsc

N = 10000
E = 320000
F = 128
NC = 2
NS = 16
NT = NC * NS
L = 16

CHUNK = 128
CPT = 80
K = 8
NB = CPT // K
PTC = CPT + K
NPAD = 10240
ROWS_PER_TILE = NPAD // NS
ZROWS = 128

_mesh = plsc.VectorSubcoreMesh(
    core_axis_name="c", subcore_axis_name="s", num_cores=NC, num_subcores=NS
)

_sc_params = pltpu.CompilerParams()
if "needs_layout_passes" in pltpu.CompilerParams.__dataclass_fields__:
    _sc_params = dataclasses.replace(_sc_params, needs_layout_passes=False)


def _spmm_sc(support, srcval2d, dst2d):
    @functools.partial(
        pl.kernel,
        out_type=(jax.ShapeDtypeStruct((NPAD, F), jnp.float32),
                  jax.ShapeDtypeStruct((NPAD, F), jnp.float32)),
        mesh=_mesh,
        compiler_params=_sc_params,
        scratch_types=[
            pltpu.VMEM_SHARED((NPAD, F), jnp.float32),
            pltpu.VMEM((2, 2 * K, 128), jnp.int32),
            pltpu.VMEM((2, K, 128), jnp.int32),
            pltpu.VMEM((2, CHUNK, F), jnp.float32),
            pltpu.SemaphoreType.DMA,
            pltpu.SemaphoreType.DMA,
            pltpu.SemaphoreType.DMA,
            pltpu.SemaphoreType.DMA,
            pltpu.SemaphoreType.DMA,
            pltpu.SemaphoreType.DMA,
            pltpu.SemaphoreType.DMA,
        ],
    )
    def k(sup_hbm, srcval_hbm, dst_hbm, outa_hbm, outb_hbm,
          acc, ebs, ebd, rows, sf0, sf1, sg0, sg1, ss0, ss1, siw):
        cid = lax.axis_index("c")
        sid = lax.axis_index("s")
        w = cid * NS + sid
        sfs = (sf0, sf1)
        sgs = (sg0, sg1)
        sss = (ss0, ss1)

        def issue_fetch(g, bb):
            pltpu.async_copy(srcval_hbm.at[pl.ds((w * PTC + g * K) * 2, 2 * K)],
                             ebs.at[bb], sfs[bb])
            pltpu.async_copy(dst_hbm.at[pl.ds(w * PTC + g * K, K)],
                             ebd.at[bb], sfs[bb])

        def wait_fetch(bb):
            pltpu.make_async_copy(srcval_hbm.at[pl.ds(0, 2 * K)],
                                  ebs.at[bb], sfs[bb]).wait()
            pltpu.make_async_copy(dst_hbm.at[pl.ds(0, K)],
                                  ebd.at[bb], sfs[bb]).wait()

        def issue_gather(bb, kk, p):
            pltpu.async_copy(sup_hbm.at[ebs.at[bb, 2 * kk]], rows.at[p], sgs[p])

        def wait_gather(p):
            pltpu.make_async_copy(sup_hbm.at[ebs.at[0, 0]],
                                  rows.at[p], sgs[p]).wait()

        def issue_scatter(bb, kk, p):
            pltpu.async_copy(rows.at[p], acc.at[ebd.at[bb, kk]], sss[p],
                             add=True)

        def wait_scatter(p):
            pltpu.make_async_copy(rows.at[p], acc.at[ebd.at[0, 0]],
                                  sss[p]).wait()

        def scale(p, bb, kk):
            @pl.loop(0, CHUNK // L)
            def _(g):
                vv = plsc.bitcast(ebs[bb, 2 * kk + 1, pl.ds(g * L, L)],
                                  jnp.float32)
                for l in range(L):
                    v = jnp.take(vv, jnp.full((L,), l, jnp.int32))
                    for j in range(F // L):
                        sl = (p, g * L + l, pl.ds(j * L, L))
                        rows[sl] = rows[sl] * v

        zvec = jnp.zeros((L,), jnp.float32)

        @pl.loop(0, ZROWS)
        def _(r):
            for j in range(F // L):
                rows[0, r, pl.ds(j * L, L)] = zvec

        for i in range(ROWS_PER_TILE // ZROWS):
            pltpu.async_copy(
                rows.at[0],
                acc.at[pl.ds(sid * ROWS_PER_TILE + i * ZROWS, ZROWS)], siw)

        issue_fetch(0, 0)

        for i in range(ROWS_PER_TILE // ZROWS):
            pltpu.make_async_copy(
                rows.at[0],
                acc.at[pl.ds(sid * ROWS_PER_TILE + i * ZROWS, ZROWS)],
                siw).wait()

        plsc.subcore_barrier()

        wait_fetch(0)
        issue_gather(0, 0, 0)

        def do_batch(g, bb, first=False):
            nbb = 1 - bb
            for kk in range(K):
                p = kk % 2
                np_ = 1 - p
                if not (first and kk == 0):
                    wait_scatter(np_)
                if kk == 0:
                    issue_fetch(g + 1, nbb)
                if kk == K - 1:
                    wait_fetch(nbb)
                    issue_gather(nbb, 0, np_)
                else:
                    issue_gather(bb, kk + 1, np_)
                wait_gather(p)
                scale(p, bb, kk)
                issue_scatter(bb, kk, p)

        do_batch(0, 0, first=True)

        @pl.loop(0, (NB - 2) // 2)
        def _(t):
            g = 1 + 2 * t
            do_batch(g, 1)
            do_batch(g + 1, 0)

        do_batch(NB - 1, 1)

        wait_scatter(1)
        wait_gather(0)

        plsc.subcore_barrier()

        for i in range(ROWS_PER_TILE // ZROWS):
            off = sid * ROWS_PER_TILE + i * ZROWS

            @pl.when(cid == 0)
            def _():
                pltpu.async_copy(acc.at[pl.ds(off, ZROWS)],
                                 outa_hbm.at[pl.ds(off, ZROWS)], siw)

            @pl.when(cid == 1)
            def _():
                pltpu.async_copy(acc.at[pl.ds(off, ZROWS)],
                                 outb_hbm.at[pl.ds(off, ZROWS)], siw)

        for i in range(ROWS_PER_TILE // ZROWS):
            off = sid * ROWS_PER_TILE + i * ZROWS

            @pl.when(cid == 0)
            def _():
                pltpu.make_async_copy(acc.at[pl.ds(off, ZROWS)],
                                      outa_hbm.at[pl.ds(off, ZROWS)],
                                      siw).wait()

            @pl.when(cid == 1)
            def _():
                pltpu.make_async_copy(acc.at[pl.ds(off, ZROWS)],
                                      outb_hbm.at[pl.ds(off, ZROWS)],
                                      siw).wait()

    return k(support, srcval2d, dst2d)


_BM = 1000


def _dot(a, b):
    return lax.dot_general(a, b, (((1,), (0,)), ((), ())),
                           precision=lax.Precision.HIGHEST,
                           preferred_element_type=jnp.float32)


def _tc_matmul(x, W):
    def body(x_ref, w_ref, o_ref):
        o_ref[...] = _dot(x_ref[...], w_ref[...])

    return pl.pallas_call(
        body,
        grid=(N // _BM,),
        in_specs=[pl.BlockSpec((_BM, F), lambda i: (i, 0)),
                  pl.BlockSpec((F, F), lambda i: (0, 0))],
        out_specs=pl.BlockSpec((_BM, F), lambda i: (i, 0)),
        out_shape=jax.ShapeDtypeStruct((N, F), jnp.float32),
    )(x, W)


def _tc_relu_matmul(pa, pb, b, W):
    def body(p0_ref, p1_ref, b_ref, w_ref, o_ref):
        h = jax.nn.relu(p0_ref[...] + p1_ref[...] + b_ref[...])
        o_ref[...] = _dot(h, w_ref[...])

    return pl.pallas_call(
        body,
        grid=(N // _BM,),
        in_specs=[pl.BlockSpec((_BM, F), lambda i: (i, 0)),
                  pl.BlockSpec((_BM, F), lambda i: (i, 0)),
                  pl.BlockSpec((1, F), lambda i: (0, 0)),
                  pl.BlockSpec((F, F), lambda i: (0, 0))],
        out_specs=pl.BlockSpec((_BM, F), lambda i: (i, 0)),
        out_shape=jax.ShapeDtypeStruct((N, F), jnp.float32),
    )(pa, pb, b.reshape(1, F), W)


def _tc_relu_bias(pa, pb, b):
    def body(p0_ref, p1_ref, b_ref, o_ref):
        o_ref[...] = jax.nn.relu(p0_ref[...] + p1_ref[...] + b_ref[...])

    return pl.pallas_call(
        body,
        grid=(N // _BM,),
        in_specs=[pl.BlockSpec((_BM, F), lambda i: (i, 0)),
                  pl.BlockSpec((_BM, F), lambda i: (i, 0)),
                  pl.BlockSpec((1, F), lambda i: (0, 0))],
        out_specs=pl.BlockSpec((_BM, F), lambda i: (i, 0)),
        out_shape=jax.ShapeDtypeStruct((N, F), jnp.float32),
    )(pa, pb, b.reshape(1, F))


def kernel(x, adj_indices, adj_values, W1, b1, W2, b2):
    dst = adj_indices[0]
    src = adj_indices[1]
    e3 = NT * CPT * CHUNK
    pad = e3 - E
    pidx = jnp.arange(pad, dtype=jnp.int32) % N
    src3 = jnp.concatenate([src, pidx]).reshape(NT, CPT, CHUNK)
    dst3 = jnp.concatenate([dst, pidx]).reshape(NT, CPT, CHUNK)
    val3 = jax.lax.bitcast_convert_type(
        jnp.concatenate([adj_values, jnp.zeros((pad,), jnp.float32)]),
        jnp.int32).reshape(NT, CPT, CHUNK)
    padw = ((0, 0), (0, PTC - CPT), (0, 0))
    src3 = jnp.pad(src3, padw)
    dst3 = jnp.pad(dst3, padw)
    val3 = jnp.pad(val3, padw)
    srcval2d = jnp.stack([src3, val3], axis=2).reshape(NT * PTC * 2, CHUNK)
    dst2d = dst3.reshape(NT * PTC, CHUNK)

    s1 = _tc_matmul(x, W1)
    p1a, p1b = _spmm_sc(s1, srcval2d, dst2d)
    s2 = _tc_relu_matmul(p1a, p1b, b1, W2)
    p2a, p2b = _spmm_sc(s2, srcval2d, dst2d)
    return _tc_relu_bias(p2a, p2b, b2)


# speedup vs baseline: 1.9466x; 1.9466x over previous
import dataclasses
import functools

import jax
import jax.numpy as jnp
from jax import lax
from jax.experimental import pallas as pl
from jax.experimental.pallas import tpu as pltpu
from jax.experimental.pallas import tpu_sc as plsc

N = 10000
E = 320000
F = 128
NC = 2
NS = 16
L = 16

CHUNK = 128
CPT = 80
CHUNKS_PER_CORE = CPT * NS
NCHUNKS = CHUNKS_PER_CORE * NC
NARR = NCHUNKS + 2 * NS
EARR = NARR * CHUNK
NPAD = 10240
ROWS_PER_TILE = NPAD // NS
ZROWS = 128

_mesh = plsc.VectorSubcoreMesh(
    core_axis_name="c", subcore_axis_name="s", num_cores=NC, num_subcores=NS
)

_sc_params = pltpu.CompilerParams()
if "needs_layout_passes" in pltpu.CompilerParams.__dataclass_fields__:
    _sc_params = dataclasses.replace(_sc_params, needs_layout_passes=False)


def _spmm_sc(support, srcval, dst):
    @functools.partial(
        pl.kernel,
        out_type=(jax.ShapeDtypeStruct((NPAD, F), jnp.float32),
                  jax.ShapeDtypeStruct((NPAD, F), jnp.float32)),
        mesh=_mesh,
        compiler_params=_sc_params,
        scratch_types=[
            pltpu.VMEM_SHARED((NPAD, F), jnp.float32),
            pltpu.VMEM((2, 2 * CHUNK), jnp.int32),
            pltpu.VMEM((2, 1, CHUNK), jnp.int32),
            pltpu.VMEM((2, CHUNK, F), jnp.float32),
            pltpu.SemaphoreType.DMA,
            pltpu.SemaphoreType.DMA,
            pltpu.SemaphoreType.DMA,
            pltpu.SemaphoreType.DMA,
            pltpu.SemaphoreType.DMA,
            pltpu.SemaphoreType.DMA,
            pltpu.SemaphoreType.DMA,
            pltpu.SemaphoreType.DMA,
        ],
    )
    def k(sup_hbm, srcval_hbm, dst_hbm, outa_hbm, outb_hbm,
          acc, srcv, dstv, rows, si0, si1, sg0, sg1, sd0, sd1, ss0, ss1):
        cid = lax.axis_index("c")
        sid = lax.axis_index("s")
        sis = (si0, si1)
        sgs = (sg0, sg1)
        sds = (sd0, sd1)
        sss = (ss0, ss1)

        def chunk_base(ti):
            return (cid * CHUNKS_PER_CORE + ti * NS + sid) * CHUNK

        def issue_srcval_fetch(ti, b):
            pltpu.async_copy(srcval_hbm.at[pl.ds(chunk_base(ti) * 2, 2 * CHUNK)],
                             srcv.at[b], sis[b])

        def wait_srcval_fetch(b):
            pltpu.make_async_copy(srcval_hbm.at[pl.ds(0, 2 * CHUNK)],
                                  srcv.at[b], sis[b]).wait()

        def issue_dst_fetch(ti, b):
            pltpu.async_copy(dst_hbm.at[pl.ds(chunk_base(ti), CHUNK)],
                             dstv.at[b, 0], sds[b])

        def wait_dst_fetch(b):
            pltpu.make_async_copy(dst_hbm.at[pl.ds(0, CHUNK)],
                                  dstv.at[b, 0], sds[b]).wait()

        def issue_gather(b):
            pltpu.async_copy(sup_hbm.at[srcv.at[b, pl.ds(0, CHUNK)]],
                             rows.at[b], sgs[b])

        def wait_gather(b):
            pltpu.make_async_copy(sup_hbm.at[srcv.at[b, pl.ds(0, CHUNK)]],
                                  rows.at[b], sgs[b]).wait()

        def issue_scatter(b):
            pltpu.async_copy(rows.at[b], acc.at[dstv.at[b, 0]], sss[b], add=True)

        def wait_scatter(b):
            pltpu.make_async_copy(rows.at[b], acc.at[dstv.at[b, 0]], sss[b]).wait()

        zvec = jnp.zeros((L,), jnp.float32)

        @pl.loop(0, ZROWS)
        def _(r):
            for j in range(F // L):
                rows[0, r, pl.ds(j * L, L)] = zvec

        for i in range(ROWS_PER_TILE // ZROWS):
            pltpu.async_copy(
                rows.at[0],
                acc.at[pl.ds(sid * ROWS_PER_TILE + i * ZROWS, ZROWS)], sg0)

        for i in range(ROWS_PER_TILE // ZROWS):
            pltpu.make_async_copy(
                rows.at[0],
                acc.at[pl.ds(sid * ROWS_PER_TILE + i * ZROWS, ZROWS)],
                sg0).wait()

        plsc.subcore_barrier()

        def scale(b):
            @pl.loop(0, CHUNK // L)
            def _(g):
                vv = plsc.bitcast(srcv[b, pl.ds(CHUNK + g * L, L)], jnp.float32)
                for l in range(L):
                    v = jnp.take(vv, jnp.full((L,), l, jnp.int32))
                    e = g * L + l
                    for j in range(F // L):
                        sl = (b, e, pl.ds(j * L, L))
                        rows[sl] = rows[sl] * v

        def half(ti, b, first=False):
            nb = 1 - b
            wait_srcval_fetch(nb)
            if not first:
                wait_scatter(nb)
            issue_dst_fetch(ti + 1, nb)
            issue_gather(nb)
            wait_gather(b)
            scale(b)
            wait_dst_fetch(b)
            issue_scatter(b)
            issue_srcval_fetch(ti + 2, b)

        issue_srcval_fetch(0, 0)
        issue_srcval_fetch(1, 1)
        wait_srcval_fetch(0)
        issue_dst_fetch(0, 0)
        issue_gather(0)

        half(0, 0, first=True)

        @pl.loop(0, (CPT - 2) // 2)
        def _(t):
            ti = 1 + 2 * t
            half(ti, 1)
            half(ti + 1, 0)

        half(CPT - 1, 1)

        wait_scatter(1)
        wait_gather(0)
        wait_dst_fetch(0)
        wait_srcval_fetch(1)

        plsc.subcore_barrier()

        for i in range(ROWS_PER_TILE // ZROWS):
            off = sid * ROWS_PER_TILE + i * ZROWS

            @pl.when(cid == 0)
            def _():
                pltpu.async_copy(acc.at[pl.ds(off, ZROWS)],
                                 outa_hbm.at[pl.ds(off, ZROWS)], sg0)

            @pl.when(cid == 1)
            def _():
                pltpu.async_copy(acc.at[pl.ds(off, ZROWS)],
                                 outb_hbm.at[pl.ds(off, ZROWS)], sg0)

        for i in range(ROWS_PER_TILE // ZROWS):
            off = sid * ROWS_PER_TILE + i * ZROWS

            @pl.when(cid == 0)
            def _():
                pltpu.make_async_copy(acc.at[pl.ds(off, ZROWS)],
                                      outa_hbm.at[pl.ds(off, ZROWS)],
                                      sg0).wait()

            @pl.when(cid == 1)
            def _():
                pltpu.make_async_copy(acc.at[pl.ds(off, ZROWS)],
                                      outb_hbm.at[pl.ds(off, ZROWS)],
                                      sg0).wait()

    return k(support, srcval, dst)


_BM = 1000


def _dot(a, b):
    return lax.dot_general(a, b, (((1,), (0,)), ((), ())),
                           precision=lax.Precision.HIGHEST,
                           preferred_element_type=jnp.float32)


def _tc_matmul(x, W):
    def body(x_ref, w_ref, o_ref):
        o_ref[...] = _dot(x_ref[...], w_ref[...])

    return pl.pallas_call(
        body,
        grid=(N // _BM,),
        in_specs=[pl.BlockSpec((_BM, F), lambda i: (i, 0)),
                  pl.BlockSpec((F, F), lambda i: (0, 0))],
        out_specs=pl.BlockSpec((_BM, F), lambda i: (i, 0)),
        out_shape=jax.ShapeDtypeStruct((N, F), jnp.float32),
    )(x, W)


def _tc_relu_matmul(pa, pb, b, W):
    def body(p0_ref, p1_ref, b_ref, w_ref, o_ref):
        h = jax.nn.relu(p0_ref[...] + p1_ref[...] + b_ref[...])
        o_ref[...] = _dot(h, w_ref[...])

    return pl.pallas_call(
        body,
        grid=(N // _BM,),
        in_specs=[pl.BlockSpec((_BM, F), lambda i: (i, 0)),
                  pl.BlockSpec((_BM, F), lambda i: (i, 0)),
                  pl.BlockSpec((1, F), lambda i: (0, 0)),
                  pl.BlockSpec((F, F), lambda i: (0, 0))],
        out_specs=pl.BlockSpec((_BM, F), lambda i: (i, 0)),
        out_shape=jax.ShapeDtypeStruct((N, F), jnp.float32),
    )(pa, pb, b.reshape(1, F), W)


def _tc_relu_bias(pa, pb, b):
    def body(p0_ref, p1_ref, b_ref, o_ref):
        o_ref[...] = jax.nn.relu(p0_ref[...] + p1_ref[...] + b_ref[...])

    return pl.pallas_call(
        body,
        grid=(N // _BM,),
        in_specs=[pl.BlockSpec((_BM, F), lambda i: (i, 0)),
                  pl.BlockSpec((_BM, F), lambda i: (i, 0)),
                  pl.BlockSpec((1, F), lambda i: (0, 0))],
        out_specs=pl.BlockSpec((_BM, F), lambda i: (i, 0)),
        out_shape=jax.ShapeDtypeStruct((N, F), jnp.float32),
    )(pa, pb, b.reshape(1, F))


def kernel(x, adj_indices, adj_values, W1, b1, W2, b2):
    dst = adj_indices[0]
    src = adj_indices[1]
    pad = EARR - E
    pidx = jnp.arange(pad, dtype=jnp.int32) % N
    src_p = jnp.concatenate([src, pidx])
    dst_p = jnp.concatenate([dst, pidx])
    vals_p = jnp.concatenate([adj_values, jnp.zeros((pad,), jnp.float32)])
    srcval = jnp.concatenate(
        [src_p.reshape(NARR, CHUNK),
         jax.lax.bitcast_convert_type(vals_p, jnp.int32).reshape(NARR, CHUNK)],
        axis=1).reshape(-1)
    s1 = _tc_matmul(x, W1)
    p1a, p1b = _spmm_sc(s1, srcval, dst_p)
    s2 = _tc_relu_matmul(p1a, p1b, b1, W2)
    p2a, p2b = _spmm_sc(s2, srcval, dst_p)
    return _tc_relu_bias(p2a, p2b, b2)
